# baseline (speedup 1.0000x reference)
import jax
import jax.numpy as jnp
from jax import lax
from jax.experimental import pallas as pl
from jax.experimental.pallas import tpu as pltpu
from jax.experimental.pallas import tpu_sc as plsc

B = 1024
NREG = 17
F = 7
P = 170
Q = P * F
XW = NREG * F
XH = 120
CELLS = 82 * 67
CTRLN = CELLS * F
CTRL_ALLOC = 38480

NC = 2
NS = 16
NW = NC * NS
NQW = 38
NCH = 75
KPAD = 176
XWIN = 24
NBCH = B // 16
H1 = 20

OFF_ROWS = 0
OFF_COLS = KPAD
OFF_RAT = 2 * KPAD
RATN = 192
COMBO_N = 2 * KPAD + RATN

BBLK = 128


def _sc_body(xt_hbm, combo_hbm, xt2_hbm, out_hbm, sem0,
             xw_v, combo_v, key_v, ctrl_v, out_v, xq_v):
    wid = lax.axis_index("s") * NC + lax.axis_index("c")
    q0 = wid * NQW

    p0 = q0 // F
    c0 = (p0 // 10) * F
    rbase = jnp.minimum(c0 - c0 % 8, XH - XWIN)
    rbase = pl.multiple_of(rbase, 8)

    xw_cp = pltpu.async_copy(xt_hbm.at[pl.ds(rbase * B, XWIN * B)], xw_v, sem0)
    pltpu.sync_copy(combo_hbm, combo_v)

    iota = lax.broadcasted_iota(jnp.int32, (16,), 0)

    for c in range(KPAD // 16):
        rws = combo_v[pl.ds(OFF_ROWS + c * 16, 16)]
        cls = combo_v[pl.ds(OFF_COLS + c * 16, 16)]
        key_v[pl.ds(c * 16, 16)] = rws * 67 + cls

    for c in range(KPAD // 16):
        keyc = key_v[pl.ds(c * 16, 16)] * F
        pid = (iota + c * 16).astype(jnp.float32)
        plsc.store_scatter(ctrl_v, [keyc], pid)
    acc = jnp.zeros((16,), jnp.int32)
    for c in range(KPAD // 16):
        keyc = key_v[pl.ds(c * 16, 16)] * F
        pid = (iota + c * 16).astype(jnp.float32)
        got = plsc.load_gather(ctrl_v, [keyc])
        valid = (iota + c * 16) < P
        acc = acc + jnp.where(jnp.logical_and(valid, got != pid), 1, 0)
    has_dup = jnp.max(acc)
    xw_cp.wait()

    nodup = has_dup == 0

    @pl.when(nodup)
    def _():
        cp = pl.multiple_of((p0 // 16) * 16, 16)
        rv0 = plsc.bitcast(combo_v[pl.ds(OFF_RAT + cp, 16)], jnp.float32)
        rv1 = plsc.bitcast(combo_v[pl.ds(OFF_RAT + cp + 16, 16)], jnp.float32)

        def one_row(qq):
            qi = q0 + qq
            pi = qi // F
            fi = qi - pi * F
            src = jnp.minimum((pi // 10) * F + fi - rbase, XWIN - 1)
            li = pi - cp
            ws = (jnp.sum(jnp.where(iota == li, rv0, 0.0))
                  + jnp.sum(jnp.where(iota == li - 16, rv1, 0.0)))
            wg = jnp.full((16,), ws)
            sbase = pl.multiple_of(src * B, 16)
            obase = pl.multiple_of(qq * B, 16)
            for j in range(NBCH):
                out_v[pl.ds(obase + j * 16, 16)] = (
                    xw_v[pl.ds(sbase + j * 16, 16)] * wg)

        @pl.when(wid < NW - 1)
        def _():
            @plsc.parallel_loop(0, H1, step=1, unroll=2)
            def _rows1(qq):
                one_row(qq)

            h1_cp = pltpu.async_copy(
                out_v.at[pl.ds(0, H1 * B)], out_hbm.at[pl.ds(q0 * B, H1 * B)],
                sem0)

            @plsc.parallel_loop(H1, NQW, step=1, unroll=2)
            def _rows2(qq):
                one_row(qq)

            h1_cp.wait()
            pltpu.sync_copy(out_v.at[pl.ds(H1 * B, (NQW - H1) * B)],
                            out_hbm.at[pl.ds((q0 + H1) * B, (NQW - H1) * B)])

        @pl.when(wid == NW - 1)
        def _():
            ntail = Q - (NW - 1) * NQW

            @plsc.parallel_loop(0, ntail, step=1, unroll=2)
            def _rows3(qq):
                one_row(qq)

            pltpu.sync_copy(out_v.at[pl.ds(0, ntail * B)],
                            out_hbm.at[pl.ds(q0 * B, ntail * B)])

    @pl.when(jnp.logical_not(nodup))
    def _():
        clo = q0 // 16
        qhi = jnp.minimum(q0 + NQW, Q)

        def chunk_idx(coff):
            qv = iota + coff
            pv = qv // F
            fv = qv - pv * F
            kv = plsc.load_gather(key_v, [pv])
            qd = kv * F + fv
            sv = (pv // 10) * F + fv
            wv = plsc.load_gather(combo_v, [OFF_RAT + pv])
            return qd, sv, plsc.bitcast(wv, jnp.float32)

        def blk_body(k, carry):
            pltpu.sync_copy(xt2_hbm.at[:, pl.ds(k * BBLK, BBLK)], xq_v)

            def bl_body(bl, carry2):
                b = k * BBLK + bl
                zero = jnp.zeros((16,), jnp.float32)
                for c in range(NCH):
                    qd, _, _ = chunk_idx(c * 16)
                    plsc.store_scatter(ctrl_v, [qd], zero)
                blv = jnp.full((16,), bl, jnp.int32)
                for c in range(NCH):
                    qd, sv, wv = chunk_idx(c * 16)
                    v = plsc.load_gather(xq_v, [sv, blv]) * wv
                    plsc.addupdate_scatter(ctrl_v, [qd], v)
                for cc in range(4):
                    coff = (clo + cc) * 16
                    qv = iota + coff
                    qd, _, _ = chunk_idx(coff)
                    got = plsc.load_gather(ctrl_v, [qd])
                    m = jnp.logical_and(qv >= q0, qv < qhi)
                    plsc.store_scatter(out_v, [(qv - q0) * B + b], got, mask=m)
                return carry2

            lax.fori_loop(0, BBLK, bl_body, 0)
            return carry

        lax.fori_loop(0, B // BBLK, blk_body, 0)

        @pl.when(wid < NW - 1)
        def _():
            pltpu.sync_copy(out_v, out_hbm.at[pl.ds(q0 * B, NQW * B)])

        @pl.when(wid == NW - 1)
        def _():
            ntail = Q - (NW - 1) * NQW
            pltpu.sync_copy(out_v.at[pl.ds(0, ntail * B)],
                            out_hbm.at[pl.ds(q0 * B, ntail * B)])


def kernel(inputs, ratios, rows, cols):
    rows_f = jnp.concatenate(
        [rows.reshape(-1).astype(jnp.int32), jnp.full((KPAD - P,), 82, jnp.int32)])
    cols_f = jnp.concatenate(
        [cols.reshape(-1).astype(jnp.int32), jnp.zeros((KPAD - P,), jnp.int32)])
    rat_f = jnp.pad(ratios.reshape(-1), (0, RATN - P)).view(jnp.int32)
    combo = jnp.concatenate([rows_f, cols_f, rat_f])

    x_t = jnp.pad(inputs, ((0, 0), (0, XH - XW))).T.reshape(-1)

    mesh = plsc.VectorSubcoreMesh(
        core_axis_name="c", subcore_axis_name="s", num_cores=NC, num_subcores=NS
    )
    run = pl.kernel(
        _sc_body,
        out_type=jax.ShapeDtypeStruct((Q * B,), jnp.float32),
        mesh=mesh,
        compiler_params=pltpu.CompilerParams(needs_layout_passes=False),
        scratch_types=[
            pltpu.SemaphoreType.DMA,
            pltpu.VMEM((XWIN * B,), jnp.float32),
            pltpu.VMEM((COMBO_N,), jnp.int32),
            pltpu.VMEM((KPAD,), jnp.int32),
            pltpu.VMEM((CTRL_ALLOC,), jnp.float32),
            pltpu.VMEM((NQW * B,), jnp.float32),
            pltpu.VMEM((XH, BBLK), jnp.float32),
        ],
    )
    out_qb = run(x_t, combo, x_t.reshape(XH, B))
    out4 = out_qb.reshape(P, F, 8, 128)
    return out4.transpose(2, 3, 0, 1).reshape(B, P, 1, F)


# speedup vs baseline: 1.2374x; 1.2374x over previous
import jax
import jax.numpy as jnp
from jax import lax
from jax.experimental import pallas as pl
from jax.experimental.pallas import tpu as pltpu
from jax.experimental.pallas import tpu_sc as plsc

B = 1024
NREG = 17
F = 7
P = 170
Q = P * F
XW = NREG * F
XH = 120
CELLS = 82 * 67
CTRLN = CELLS * F
CTRL_ALLOC = 38480

NC = 2
NS = 16
NW = NC * NS
NQW = 38
NCH = 75
KPAD = 176
XWIN = 24
NBCH = B // 16
H1 = 20

OFF_ROWS = 0
OFF_COLS = KPAD
OFF_RAT = 2 * KPAD
RATN = 192
COMBO_N = 2 * KPAD + RATN

BBLK = 128


def _sc_body(xt_hbm, combo_hbm, xt2_hbm, out_hbm, sem0,
             xw_v, combo_v, key_v, ctrl_v, out_v, xq_v):
    wid = lax.axis_index("s") * NC + lax.axis_index("c")
    q0 = wid * NQW

    p0 = q0 // F
    c0 = (p0 // 10) * F
    rbase = jnp.minimum(c0 - c0 % 8, XH - XWIN)
    rbase = pl.multiple_of(rbase, 8)

    xw_cp = pltpu.async_copy(xt_hbm.at[pl.ds(rbase * B, XWIN * B)], xw_v, sem0)
    pltpu.sync_copy(combo_hbm, combo_v)

    iota = lax.broadcasted_iota(jnp.int32, (16,), 0)

    def key_c(c, cy):
        c16 = pl.multiple_of(c * 16, 16)
        rws = combo_v[pl.ds(OFF_ROWS + c16, 16)]
        cls = combo_v[pl.ds(OFF_COLS + c16, 16)]
        kv = rws * 67 + cls
        key_v[pl.ds(c16, 16)] = kv
        plsc.store_scatter(ctrl_v, [kv * F], (iota + c16).astype(jnp.float32))
        return cy

    lax.fori_loop(0, KPAD // 16, key_c, 0)

    def chk_c(c, accum):
        c16 = pl.multiple_of(c * 16, 16)
        keyc = key_v[pl.ds(c16, 16)] * F
        pid = (iota + c16).astype(jnp.float32)
        got = plsc.load_gather(ctrl_v, [keyc])
        valid = (iota + c16) < P
        return accum + jnp.where(jnp.logical_and(valid, got != pid), 1, 0)

    acc = lax.fori_loop(0, KPAD // 16, chk_c, jnp.zeros((16,), jnp.int32))
    has_dup = jnp.max(acc)
    xw_cp.wait()

    nodup = has_dup == 0

    @pl.when(nodup)
    def _():
        cp = pl.multiple_of((p0 // 16) * 16, 16)
        rv0 = plsc.bitcast(combo_v[pl.ds(OFF_RAT + cp, 16)], jnp.float32)
        rv1 = plsc.bitcast(combo_v[pl.ds(OFF_RAT + cp + 16, 16)], jnp.float32)

        def one_row(qq):
            qi = q0 + qq
            pi = qi // F
            fi = qi - pi * F
            src = jnp.minimum((pi // 10) * F + fi - rbase, XWIN - 1)
            li = pi - cp
            ws = (jnp.sum(jnp.where(iota == li, rv0, 0.0))
                  + jnp.sum(jnp.where(iota == li - 16, rv1, 0.0)))
            wg = jnp.full((16,), ws)
            sbase = pl.multiple_of(src * B, 16)
            obase = pl.multiple_of(qq * B, 16)
            for j in range(NBCH):
                out_v[pl.ds(obase + j * 16, 16)] = (
                    xw_v[pl.ds(sbase + j * 16, 16)] * wg)

        @plsc.parallel_loop(0, NQW, step=1, unroll=1)
        def _rows(qq):
            one_row(qq)

        @pl.when(wid < NW - 1)
        def _():
            pltpu.sync_copy(out_v, out_hbm.at[pl.ds(q0 * B, NQW * B)])

        @pl.when(wid == NW - 1)
        def _():
            ntail = Q - (NW - 1) * NQW
            pltpu.sync_copy(out_v.at[pl.ds(0, ntail * B)],
                            out_hbm.at[pl.ds(q0 * B, ntail * B)])

    @pl.when(jnp.logical_not(nodup))
    def _():
        clo = q0 // 16
        qhi = jnp.minimum(q0 + NQW, Q)

        def chunk_idx(coff):
            qv = iota + coff
            pv = qv // F
            fv = qv - pv * F
            kv = plsc.load_gather(key_v, [pv])
            qd = kv * F + fv
            sv = (pv // 10) * F + fv
            wv = plsc.load_gather(combo_v, [OFF_RAT + pv])
            return qd, sv, plsc.bitcast(wv, jnp.float32)

        def blk_body(k, carry):
            pltpu.sync_copy(xt2_hbm.at[:, pl.ds(k * BBLK, BBLK)], xq_v)

            def bl_body(bl, carry2):
                b = k * BBLK + bl
                zero = jnp.zeros((16,), jnp.float32)

                def zero_c(c, cy):
                    qd, _, _ = chunk_idx(c * 16)
                    plsc.store_scatter(ctrl_v, [qd], zero)
                    return cy

                lax.fori_loop(0, NCH, zero_c, 0)
                blv = jnp.full((16,), bl, jnp.int32)

                def add_c(c, cy):
                    qd, sv, wv = chunk_idx(c * 16)
                    v = plsc.load_gather(xq_v, [sv, blv]) * wv
                    plsc.addupdate_scatter(ctrl_v, [qd], v)
                    return cy

                lax.fori_loop(0, NCH, add_c, 0)

                def get_c(cc, cy):
                    coff = (clo + cc) * 16
                    qv = iota + coff
                    qd, _, _ = chunk_idx(coff)
                    got = plsc.load_gather(ctrl_v, [qd])
                    m = jnp.logical_and(qv >= q0, qv < qhi)
                    plsc.store_scatter(out_v, [(qv - q0) * B + b], got, mask=m)
                    return cy

                lax.fori_loop(0, 4, get_c, 0)
                return carry2

            lax.fori_loop(0, BBLK, bl_body, 0)
            return carry

        lax.fori_loop(0, B // BBLK, blk_body, 0)

        @pl.when(wid < NW - 1)
        def _():
            pltpu.sync_copy(out_v, out_hbm.at[pl.ds(q0 * B, NQW * B)])

        @pl.when(wid == NW - 1)
        def _():
            ntail = Q - (NW - 1) * NQW
            pltpu.sync_copy(out_v.at[pl.ds(0, ntail * B)],
                            out_hbm.at[pl.ds(q0 * B, ntail * B)])


def kernel(inputs, ratios, rows, cols):
    rows_f = jnp.concatenate(
        [rows.reshape(-1).astype(jnp.int32), jnp.full((KPAD - P,), 82, jnp.int32)])
    cols_f = jnp.concatenate(
        [cols.reshape(-1).astype(jnp.int32), jnp.zeros((KPAD - P,), jnp.int32)])
    rat_f = jnp.pad(ratios.reshape(-1), (0, RATN - P)).view(jnp.int32)
    combo = jnp.concatenate([rows_f, cols_f, rat_f])

    x_t = jnp.pad(inputs, ((0, 0), (0, XH - XW))).T.reshape(-1)

    mesh = plsc.VectorSubcoreMesh(
        core_axis_name="c", subcore_axis_name="s", num_cores=NC, num_subcores=NS
    )
    run = pl.kernel(
        _sc_body,
        out_type=jax.ShapeDtypeStruct((Q * B,), jnp.float32),
        mesh=mesh,
        compiler_params=pltpu.CompilerParams(needs_layout_passes=False),
        scratch_types=[
            pltpu.SemaphoreType.DMA,
            pltpu.VMEM((XWIN * B,), jnp.float32),
            pltpu.VMEM((COMBO_N,), jnp.int32),
            pltpu.VMEM((KPAD,), jnp.int32),
            pltpu.VMEM((CTRL_ALLOC,), jnp.float32),
            pltpu.VMEM((NQW * B,), jnp.float32),
            pltpu.VMEM((XH, BBLK), jnp.float32),
        ],
    )
    out_qb = run(x_t, combo, x_t.reshape(XH, B))
    out4 = out_qb.reshape(P, F, 8, 128)
    return out4.transpose(2, 3, 0, 1).reshape(B, P, 1, F)
